# baseline (speedup 1.0000x reference)
import functools

import jax
import jax.numpy as jnp
from jax import lax
from jax.experimental import pallas as pl
from jax.experimental.pallas import tpu as pltpu
from jax.experimental.pallas import tpu_sc as plsc

VOCAB_ = 1000000
D = 64
NC = 2
NS = 16
NW = NC * NS


def _make_sc_gather(B: int, C: int):
    b_per_w = B // NW
    nchunk = b_per_w // C
    mesh = plsc.VectorSubcoreMesh(core_axis_name="c", subcore_axis_name="s")

    @functools.partial(
        pl.kernel,
        out_type=jax.ShapeDtypeStruct((B, D), jnp.float32),
        mesh=mesh,
        scratch_types=[
            pltpu.VMEM((C,), jnp.int32),
            pltpu.VMEM((C, D), jnp.float32),
            pltpu.SemaphoreType.DMA,
        ],
        compiler_params=pltpu.CompilerParams(use_tc_tiling_on_sc=False),
    )
    def sc_gather(table_hbm, idx_hbm, out_hbm, idx_v, rows_v, sem):
        wid = lax.axis_index("s") * NC + lax.axis_index("c")
        wbase = wid * b_per_w

        @pl.loop(0, nchunk)
        def _(g):
            base = pl.multiple_of(wbase + g * C, 8)
            pltpu.sync_copy(idx_hbm.at[pl.ds(base, C)], idx_v)
            pltpu.async_copy(table_hbm.at[idx_v], rows_v, sem).wait()
            pltpu.sync_copy(rows_v, out_hbm.at[pl.ds(base, C)])

    return sc_gather


def kernel(tokens, embed_weight):
    S, T = tokens.shape
    B = S * T
    flat = tokens.reshape(B)
    out = _make_sc_gather(B, 1024)(embed_weight, flat)
    return out.reshape(S, T, D)


# speedup vs baseline: 1.0250x; 1.0250x over previous
import functools

import jax
import jax.numpy as jnp
from jax import lax
from jax.experimental import pallas as pl
from jax.experimental.pallas import tpu as pltpu
from jax.experimental.pallas import tpu_sc as plsc

D = 64
NC = 2
NS = 16
NW = NC * NS


def _make_sc_gather(B: int, C: int):
    b_per_w = B // NW
    nchunk = b_per_w // C
    assert nchunk >= 4 and nchunk % 2 == 0 and b_per_w % C == 0 and C % 8 == 0
    mesh = plsc.VectorSubcoreMesh(core_axis_name="c", subcore_axis_name="s")

    @functools.partial(
        pl.kernel,
        out_type=jax.ShapeDtypeStruct((B, D), jnp.float32),
        mesh=mesh,
        scratch_types=[
            pltpu.VMEM((C,), jnp.int32),
            pltpu.VMEM((C,), jnp.int32),
            pltpu.VMEM((C, D), jnp.float32),
            pltpu.VMEM((C, D), jnp.float32),
            pltpu.SemaphoreType.DMA,
            pltpu.SemaphoreType.DMA,
            pltpu.SemaphoreType.DMA,
            pltpu.SemaphoreType.DMA,
            pltpu.SemaphoreType.DMA,
            pltpu.SemaphoreType.DMA,
        ],
        compiler_params=pltpu.CompilerParams(use_tc_tiling_on_sc=False),
    )
    def sc_gather(table_hbm, idx_hbm, out_hbm, idx0, idx1, rows0, rows1,
                  si0, si1, sg0, sg1, ss0, ss1):
        idxs = (idx0, idx1)
        rows = (rows0, rows1)
        si = (si0, si1)
        sg = (sg0, sg1)
        ss = (ss0, ss1)
        wid = lax.axis_index("s") * NC + lax.axis_index("c")
        wbase = pl.multiple_of(wid * b_per_w, 8)

        def idx_desc(g, b):
            base = pl.multiple_of(wbase + g * C, 8)
            return pltpu.make_async_copy(
                idx_hbm.at[pl.ds(base, C)], idxs[b], si[b])

        def gather_desc(g, b):
            return pltpu.make_async_copy(
                table_hbm.at[idxs[b]], rows[b], sg[b])

        def scatter_desc(g, b):
            base = pl.multiple_of(wbase + g * C, 8)
            return pltpu.make_async_copy(
                rows[b], out_hbm.at[pl.ds(base, C)], ss[b])

        idx_desc(0, 0).start()
        idx_desc(1, 1).start()
        idx_desc(0, 0).wait()
        gather_desc(0, 0).start()
        idx_desc(1, 1).wait()
        gather_desc(1, 1).start()
        gather_desc(0, 0).wait()
        scatter_desc(0, 0).start()
        idx_desc(2, 0).start()

        @pl.loop(2, nchunk, step=2)
        def _(g0):
            for b in (0, 1):
                g = g0 + b
                scatter_desc(g - 2, b).wait()
                idx_desc(g, b).wait()
                gather_desc(g, b).start()
                gather_desc(g - 1, 1 - b).wait()
                scatter_desc(g - 1, 1 - b).start()
                g_next = jnp.minimum(g + 1, nchunk - 1)
                idx_desc(g_next, 1 - b).start()

        gather_desc(nchunk - 1, 1).wait()
        scatter_desc(nchunk - 1, 1).start()
        idx_desc(nchunk - 1, 0).wait()
        scatter_desc(nchunk - 2, 0).wait()
        scatter_desc(nchunk - 1, 1).wait()

    return sc_gather


def kernel(tokens, embed_weight):
    S, T = tokens.shape
    B = S * T
    flat = tokens.reshape(B)
    out = _make_sc_gather(B, 512)(embed_weight, flat)
    return out.reshape(S, T, D)
